# Initial kernel scaffold; baseline (speedup 1.0000x reference)
import jax
import jax.numpy as jnp
from jax.experimental import pallas as pl


def kernel(input_ids, token_type_ids, W_word, W_pos, W_tok, gamma, beta):
    raise NotImplementedError("write your pallas kernel here")


# speedup vs baseline: 2.7966x; 2.7966x over previous
import functools

import jax
import jax.numpy as jnp
from jax import lax
from jax.experimental import pallas as pl
from jax.experimental.pallas import tpu as pltpu
from jax.experimental.pallas import tpu_sc as plsc

_EPS = 1e-12


_NW = 32
_ROWS_PER_GATHER = 128


def _sc_gather(W_word, ids_grouped):
    nw, nsub, rg = ids_grouped.shape
    D = W_word.shape[1]
    n = nw * nsub * rg
    per_w = nsub * rg
    mesh = plsc.VectorSubcoreMesh(core_axis_name="c", subcore_axis_name="s")

    @functools.partial(
        pl.kernel,
        mesh=mesh,
        out_type=jax.ShapeDtypeStruct((n, D), jnp.float32),
        scratch_types=[
            pltpu.VMEM((nsub, rg), jnp.int32),
            pltpu.VMEM((rg, D), jnp.float32),
            pltpu.VMEM((rg, D), jnp.float32),
            pltpu.SemaphoreType.DMA,
            pltpu.SemaphoreType.DMA,
        ],
    )
    def k(table_hbm, idx_hbm, out_hbm, idx_v, buf0, buf1, sem0, sem1):
        wid = lax.axis_index("s") * 2 + lax.axis_index("c")
        base = wid * per_w
        pltpu.sync_copy(idx_hbm.at[wid], idx_v)
        bufs = (buf0, buf1)
        sems = (sem0, sem1)
        descs = [None] * nsub
        descs[0] = pltpu.async_copy(table_hbm.at[idx_v.at[0]], buf0, sem0)
        for j in range(nsub):
            if j + 1 < nsub:
                descs[j + 1] = pltpu.async_copy(
                    table_hbm.at[idx_v.at[j + 1]], bufs[(j + 1) % 2],
                    sems[(j + 1) % 2])
            descs[j].wait()
            pltpu.sync_copy(bufs[j % 2], out_hbm.at[pl.ds(base + j * rg, rg)])

    return k(W_word, ids_grouped)


def _ln_body(g_ref, pos_ref, tt_ref, wtok_ref, gamma_ref, beta_ref, out_ref):
    x = g_ref[...]
    pos = pos_ref[...]
    tt = tt_ref[...]
    t0 = wtok_ref[0:1, :]
    t1 = wtok_ref[1:2, :]
    tok = jnp.where(tt != 0, t1, t0)
    e = x + pos + tok
    mean = jnp.mean(e, axis=1, keepdims=True)
    c = e - mean
    var = jnp.mean(c * c, axis=1, keepdims=True)
    inv = lax.rsqrt(var + _EPS)
    out_ref[...] = c * inv * gamma_ref[...] + beta_ref[...]


def _ln_fuse(gathered, W_pos, tt_flat, W_tok, gamma, beta, blk):
    n, D = gathered.shape
    S = W_pos.shape[0]
    pos_blocks = S // blk
    grid = (n // blk,)
    return pl.pallas_call(
        _ln_body,
        grid=grid,
        in_specs=[
            pl.BlockSpec((blk, D), lambda i: (i, 0)),
            pl.BlockSpec((blk, D), lambda i: (i % pos_blocks, 0)),
            pl.BlockSpec((blk, 1), lambda i: (i, 0)),
            pl.BlockSpec((2, D), lambda i: (0, 0)),
            pl.BlockSpec((1, D), lambda i: (0, 0)),
            pl.BlockSpec((1, D), lambda i: (0, 0)),
        ],
        out_specs=pl.BlockSpec((blk, D), lambda i: (i, 0)),
        out_shape=jax.ShapeDtypeStruct((n, D), jnp.float32),
    )(gathered, W_pos, tt_flat, W_tok, gamma, beta)


def kernel(input_ids, token_type_ids, W_word, W_pos, W_tok, gamma, beta):
    B, S = input_ids.shape
    D = W_word.shape[1]
    n = B * S
    ids_grouped = input_ids.astype(jnp.int32).reshape(
        _NW, n // (_NW * _ROWS_PER_GATHER), _ROWS_PER_GATHER)
    gathered = _sc_gather(W_word, ids_grouped)
    tt_flat = token_type_ids.astype(jnp.int32).reshape(n, 1)
    out = _ln_fuse(gathered, W_pos, tt_flat, W_tok,
                   gamma.reshape(1, D), beta.reshape(1, D), blk=2048)
    return out.reshape(B, S, D)
